# baseline (speedup 1.0000x reference)
import functools

import jax
import jax.numpy as jnp
from jax.experimental import pallas as pl
from jax.experimental.pallas import tpu as pltpu

_BM = 512
_PAR = pltpu.CompilerParams(dimension_semantics=("parallel",))


def _hop_cast_body(a_ref, x_ref, abf_ref, xbf_ref, o_ref):
    a_bf = a_ref[...].astype(jnp.bfloat16)
    abf_ref[...] = a_bf
    x_bf = x_ref[...].astype(jnp.bfloat16)

    @pl.when(pl.program_id(0) == 0)
    def _():
        xbf_ref[...] = x_bf

    o_ref[...] = jnp.dot(
        a_bf, x_bf, preferred_element_type=jnp.float32
    ).astype(jnp.bfloat16)


def _hop_cast(a_f32, x_f32, bm=256):
    n, d = x_f32.shape
    bm = min(bm, n)
    return pl.pallas_call(
        _hop_cast_body,
        grid=(n // bm,),
        in_specs=[
            pl.BlockSpec((bm, n), lambda i: (i, 0)),
            pl.BlockSpec((n, d), lambda i: (0, 0)),
        ],
        out_specs=[
            pl.BlockSpec((bm, n), lambda i: (i, 0)),
            pl.BlockSpec((n, d), lambda i: (0, 0)),
            pl.BlockSpec((bm, d), lambda i: (i, 0)),
        ],
        out_shape=[
            jax.ShapeDtypeStruct((n, n), jnp.bfloat16),
            jax.ShapeDtypeStruct((n, d), jnp.bfloat16),
            jax.ShapeDtypeStruct((n, d), jnp.bfloat16),
        ],
        compiler_params=_PAR,
    )(a_f32, x_f32)


def _hop_body(a_ref, x_ref, o_ref, acc_ref):
    k = pl.program_id(1)
    part = jnp.dot(a_ref[...], x_ref[...], preferred_element_type=jnp.float32)

    @pl.when(k == 0)
    def _():
        acc_ref[...] = part

    @pl.when(k == 1)
    def _():
        o_ref[...] = (acc_ref[...] + part).astype(jnp.bfloat16)


def _hop(a_bf, x_bf, bm=1024):
    n, d = x_bf.shape
    bm = min(bm, n)
    bk = n // 2
    return pl.pallas_call(
        _hop_body,
        grid=(n // bm, 2),
        in_specs=[
            pl.BlockSpec((bm, bk), lambda i, k: (i, k)),
            pl.BlockSpec((bk, d), lambda i, k: (k, 0)),
        ],
        out_specs=pl.BlockSpec((bm, d), lambda i, k: (i, 0)),
        out_shape=jax.ShapeDtypeStruct((n, d), jnp.bfloat16),
        scratch_shapes=[pltpu.VMEM((bm, d), jnp.float32)],
        compiler_params=pltpu.CompilerParams(
            dimension_semantics=("parallel", "arbitrary")
        ),
    )(a_bf, x_bf)


def _bn_scale_shift(st_ref, g_ref, bt_ref, n):
    m = st_ref[0:1, :] / n
    v = st_ref[1:2, :] / n - m * m
    scale = g_ref[...] * jax.lax.rsqrt(v + 1e-5)
    shift = bt_ref[...] - m * scale
    return scale, shift


def _tail_body(
    x0_ref, x1_ref, x2_ref, x3_ref, w1_ref, b1_ref, g1_ref, bt1_ref,
    w2_ref, b2_ref, g2_ref, bt2_ref, wout_ref, bout_ref, idx_ref,
    out_ref, z_ref, st1_ref, st2_ref, ge_ref, *, n, ng, bm,
):
    p = pl.program_id(0)
    i = pl.program_id(1)
    nb = pl.num_programs(1)
    rows = pl.ds(i * bm, bm)

    @pl.when(p == 0)
    def _phase_a():
        acc = jnp.dot(x0_ref[...], w1_ref[0], preferred_element_type=jnp.float32)
        acc += jnp.dot(x1_ref[...], w1_ref[1], preferred_element_type=jnp.float32)
        acc += jnp.dot(x2_ref[...], w1_ref[2], preferred_element_type=jnp.float32)
        acc += jnp.dot(x3_ref[...], w1_ref[3], preferred_element_type=jnp.float32)
        z = acc + b1_ref[...]
        z_ref[rows, :] = z
        st = jnp.stack([jnp.sum(z, axis=0), jnp.sum(z * z, axis=0)])

        @pl.when(i == 0)
        def _():
            st1_ref[...] = st

        @pl.when(i > 0)
        def _():
            st1_ref[...] += st

    @pl.when(p == 1)
    def _phase_b():
        scale, shift = _bn_scale_shift(st1_ref, g1_ref, bt1_ref, n)
        h = jnp.maximum(z_ref[rows, :] * scale + shift, 0.0)
        z2 = (
            jnp.dot(h.astype(jnp.bfloat16), w2_ref[...],
                    preferred_element_type=jnp.float32)
            + b2_ref[...]
        )
        z_ref[rows, :] = z2
        st = jnp.stack([jnp.sum(z2, axis=0), jnp.sum(z2 * z2, axis=0)])

        @pl.when(i == 0)
        def _():
            st2_ref[...] = st

        @pl.when(i > 0)
        def _():
            st2_ref[...] += st

    @pl.when(p == 2)
    def _phase_c():
        scale, shift = _bn_scale_shift(st2_ref, g2_ref, bt2_ref, n)
        h = jnp.maximum(z_ref[rows, :] * scale + shift, 0.0)
        idv = idx_ref[0, 0, :]
        gid = jax.lax.broadcasted_iota(jnp.int32, (ng, bm), 0)
        onehot = (gid == idv[None, :]).astype(jnp.float32)
        part = jnp.dot(onehot, h, preferred_element_type=jnp.float32,
                       precision=jax.lax.Precision.HIGHEST)

        @pl.when(i == 0)
        def _():
            ge_ref[...] = part

        @pl.when(i > 0)
        def _():
            ge_ref[...] += part

        @pl.when(i == nb - 1)
        def _():
            out_ref[...] = (
                jax.lax.dot_general(
                    wout_ref[...], ge_ref[...],
                    (((1,), (1,)), ((), ())),
                    preferred_element_type=jnp.float32,
                )
                + bout_ref[...]
            )


def _tail(x0, x1, x2, x3, w1s, b1r, g1r, bt1r, w2b, b2r, g2r, bt2r,
          woutr, boutr, idx3, ng):
    n, d = x0.shape
    hid = w2b.shape[0]
    bm = min(4 * _BM, n)

    def xmap(p, i):
        return (jnp.where(p == 0, i, 0), 0)

    xspec = pl.BlockSpec((bm, d), xmap)
    vspec = pl.BlockSpec((1, hid), lambda p, i: (0, 0))
    return pl.pallas_call(
        functools.partial(_tail_body, n=n, ng=ng, bm=bm),
        grid=(3, n // bm),
        in_specs=[
            xspec, xspec, xspec, xspec,
            pl.BlockSpec((4, d, hid), lambda p, i: (0, 0, 0)),
            vspec, vspec, vspec,
            pl.BlockSpec((hid, hid), lambda p, i: (0, 0)),
            vspec, vspec, vspec, vspec,
            pl.BlockSpec((1, ng), lambda p, i: (0, 0)),
            pl.BlockSpec((1, 1, bm), lambda p, i: (jnp.where(p == 2, i, 0), 0, 0)),
        ],
        out_specs=pl.BlockSpec((1, ng), lambda p, i: (0, 0)),
        out_shape=jax.ShapeDtypeStruct((1, ng), jnp.float32),
        scratch_shapes=[
            pltpu.VMEM((n, hid), jnp.float32),
            pltpu.VMEM((2, hid), jnp.float32),
            pltpu.VMEM((2, hid), jnp.float32),
            pltpu.VMEM((ng, hid), jnp.float32),
        ],
    )(x0, x1, x2, x3, w1s, b1r, g1r, bt1r, w2b, b2r, g2r, bt2r,
      woutr, boutr, idx3)


def kernel(A, X, idx, W1, b1, g1, bt1, W2, b2, g2, bt2, Wout, bout):
    n, d = X.shape
    hid = W2.shape[0]
    ng = 64
    tbm = min(4 * _BM, n)

    a_bf, x0, x1 = _hop_cast(A, X)
    x2 = _hop(a_bf, x1)
    x3 = _hop(a_bf, x2)

    w1s = W1.reshape(4, d, hid).astype(jnp.bfloat16)
    pooled = _tail(
        x0, x1, x2, x3, w1s,
        b1.reshape(1, hid), g1.reshape(1, hid), bt1.reshape(1, hid),
        W2.astype(jnp.bfloat16), b2.reshape(1, hid),
        g2.reshape(1, hid), bt2.reshape(1, hid),
        Wout.reshape(1, hid), jnp.broadcast_to(bout.reshape(1, 1), (1, ng)),
        idx.reshape(n // tbm, 1, tbm), ng,
    )
    return pooled[0]


# speedup vs baseline: 1.0641x; 1.0641x over previous
import functools

import jax
import jax.numpy as jnp
from jax.experimental import pallas as pl
from jax.experimental.pallas import tpu as pltpu

_BM = 512
_PAR = pltpu.CompilerParams(dimension_semantics=("parallel",))


def _hop_cast_body(a_ref, x_ref, abf_ref, xbf_ref, o_ref):
    a_bf = a_ref[...].astype(jnp.bfloat16)
    abf_ref[...] = a_bf
    x_bf = x_ref[...].astype(jnp.bfloat16)

    @pl.when(pl.program_id(0) == 0)
    def _():
        xbf_ref[...] = x_bf

    o_ref[...] = jnp.dot(
        a_bf, x_bf, preferred_element_type=jnp.float32
    ).astype(jnp.bfloat16)


def _hop_cast(a_f32, x_f32, bm=256):
    n, d = x_f32.shape
    bm = min(bm, n)
    return pl.pallas_call(
        _hop_cast_body,
        grid=(n // bm,),
        in_specs=[
            pl.BlockSpec((bm, n), lambda i: (i, 0)),
            pl.BlockSpec((n, d), lambda i: (0, 0)),
        ],
        out_specs=[
            pl.BlockSpec((bm, n), lambda i: (i, 0)),
            pl.BlockSpec((n, d), lambda i: (0, 0)),
            pl.BlockSpec((bm, d), lambda i: (i, 0)),
        ],
        out_shape=[
            jax.ShapeDtypeStruct((n, n), jnp.bfloat16),
            jax.ShapeDtypeStruct((n, d), jnp.bfloat16),
            jax.ShapeDtypeStruct((n, d), jnp.bfloat16),
        ],
        compiler_params=_PAR,
    )(a_f32, x_f32)


def _hop_body(a_ref, x_ref, o_ref):
    o_ref[...] = jnp.dot(
        a_ref[...], x_ref[...], preferred_element_type=jnp.float32
    ).astype(jnp.bfloat16)


def _hop(a_bf, x_bf, bm=1024):
    n, d = x_bf.shape
    bm = min(bm, n)
    return pl.pallas_call(
        _hop_body,
        grid=(n // bm,),
        in_specs=[
            pl.BlockSpec((bm, n), lambda i: (i, 0)),
            pl.BlockSpec((n, d), lambda i: (0, 0)),
        ],
        out_specs=pl.BlockSpec((bm, d), lambda i: (i, 0)),
        out_shape=jax.ShapeDtypeStruct((n, d), jnp.bfloat16),
        compiler_params=_PAR,
    )(a_bf, x_bf)


def _bn_scale_shift(st_ref, g_ref, bt_ref, n):
    m = st_ref[0:1, :] / n
    v = st_ref[1:2, :] / n - m * m
    scale = g_ref[...] * jax.lax.rsqrt(v + 1e-5)
    shift = bt_ref[...] - m * scale
    return scale, shift


def _tail_body(
    x0_ref, x1_ref, x2_ref, x3_ref, w1_ref, b1_ref, g1_ref, bt1_ref,
    w2_ref, b2_ref, g2_ref, bt2_ref, wout_ref, bout_ref, idx_ref,
    out_ref, z_ref, st1_ref, st2_ref, ge_ref, *, n, ng, bm,
):
    p = pl.program_id(0)
    i = pl.program_id(1)
    nb = pl.num_programs(1)
    rows = pl.ds(i * bm, bm)

    @pl.when(p == 0)
    def _phase_a():
        hcat = jnp.concatenate(
            [x0_ref[...], x1_ref[...], x2_ref[...], x3_ref[...]], axis=1
        )
        z = (
            jnp.dot(hcat, w1_ref[...], preferred_element_type=jnp.float32)
            + b1_ref[...]
        )
        z_ref[rows, :] = z
        st = jnp.stack([jnp.sum(z, axis=0), jnp.sum(z * z, axis=0)])

        @pl.when(i == 0)
        def _():
            st1_ref[...] = st

        @pl.when(i > 0)
        def _():
            st1_ref[...] += st

    @pl.when(p == 1)
    def _phase_b():
        scale, shift = _bn_scale_shift(st1_ref, g1_ref, bt1_ref, n)
        h = jnp.maximum(z_ref[rows, :] * scale + shift, 0.0)
        z2 = (
            jnp.dot(h.astype(jnp.bfloat16), w2_ref[...],
                    preferred_element_type=jnp.float32)
            + b2_ref[...]
        )
        z_ref[rows, :] = z2
        st = jnp.stack([jnp.sum(z2, axis=0), jnp.sum(z2 * z2, axis=0)])

        @pl.when(i == 0)
        def _():
            st2_ref[...] = st

        @pl.when(i > 0)
        def _():
            st2_ref[...] += st

    @pl.when(p == 2)
    def _phase_c():
        scale, shift = _bn_scale_shift(st2_ref, g2_ref, bt2_ref, n)
        h = jnp.maximum(z_ref[rows, :] * scale + shift, 0.0)
        idv = idx_ref[0, 0, :]
        gid = jax.lax.broadcasted_iota(jnp.int32, (ng, bm), 0)
        onehot = (gid == idv[None, :]).astype(jnp.float32)
        part = jnp.dot(onehot, h, preferred_element_type=jnp.float32,
                       precision=jax.lax.Precision.HIGHEST)

        @pl.when(i == 0)
        def _():
            ge_ref[...] = part

        @pl.when(i > 0)
        def _():
            ge_ref[...] += part

        @pl.when(i == nb - 1)
        def _():
            out_ref[...] = (
                jax.lax.dot_general(
                    wout_ref[...], ge_ref[...],
                    (((1,), (1,)), ((), ())),
                    preferred_element_type=jnp.float32,
                )
                + bout_ref[...]
            )


def _tail(x0, x1, x2, x3, w1s, b1r, g1r, bt1r, w2b, b2r, g2r, bt2r,
          woutr, boutr, idx3, ng):
    n, d = x0.shape
    hid = w2b.shape[0]
    bm = min(4 * _BM, n)

    def xmap(p, i):
        return (jnp.where(p == 0, i, 0), 0)

    xspec = pl.BlockSpec((bm, d), xmap)
    vspec = pl.BlockSpec((1, hid), lambda p, i: (0, 0))
    return pl.pallas_call(
        functools.partial(_tail_body, n=n, ng=ng, bm=bm),
        grid=(3, n // bm),
        in_specs=[
            xspec, xspec, xspec, xspec,
            pl.BlockSpec((4 * d, hid), lambda p, i: (0, 0)),
            vspec, vspec, vspec,
            pl.BlockSpec((hid, hid), lambda p, i: (0, 0)),
            vspec, vspec, vspec, vspec,
            pl.BlockSpec((1, ng), lambda p, i: (0, 0)),
            pl.BlockSpec((1, 1, bm), lambda p, i: (jnp.where(p == 2, i, 0), 0, 0)),
        ],
        out_specs=pl.BlockSpec((1, ng), lambda p, i: (0, 0)),
        out_shape=jax.ShapeDtypeStruct((1, ng), jnp.float32),
        scratch_shapes=[
            pltpu.VMEM((n, hid), jnp.float32),
            pltpu.VMEM((2, hid), jnp.float32),
            pltpu.VMEM((2, hid), jnp.float32),
            pltpu.VMEM((ng, hid), jnp.float32),
        ],
    )(x0, x1, x2, x3, w1s, b1r, g1r, bt1r, w2b, b2r, g2r, bt2r,
      woutr, boutr, idx3)


def kernel(A, X, idx, W1, b1, g1, bt1, W2, b2, g2, bt2, Wout, bout):
    n, d = X.shape
    hid = W2.shape[0]
    ng = 64
    tbm = min(4 * _BM, n)

    a_bf, x0, x1 = _hop_cast(A, X)
    x2 = _hop(a_bf, x1)
    x3 = _hop(a_bf, x2)

    w1s = W1.astype(jnp.bfloat16)
    pooled = _tail(
        x0, x1, x2, x3, w1s,
        b1.reshape(1, hid), g1.reshape(1, hid), bt1.reshape(1, hid),
        W2.astype(jnp.bfloat16), b2.reshape(1, hid),
        g2.reshape(1, hid), bt2.reshape(1, hid),
        Wout.reshape(1, hid), jnp.broadcast_to(bout.reshape(1, 1), (1, ng)),
        idx.reshape(n // tbm, 1, tbm), ng,
    )
    return pooled[0]
